# Initial kernel scaffold; baseline (speedup 1.0000x reference)
import jax
import jax.numpy as jnp
from jax.experimental import pallas as pl


def kernel(loc, conf, target_boxes, target_labels):
    raise NotImplementedError("write your pallas kernel here")


# speedup vs baseline: 449.1893x; 449.1893x over previous
import functools

import jax
import jax.numpy as jnp
import numpy as np
from jax import lax
from jax.experimental import pallas as pl
from jax.experimental.pallas import tpu as pltpu
from jax.experimental.pallas import tpu_sc as plsc

N = 5000
NPAD = 5120
NCLS = 21
CHUNKS = NPAD // 16
NEG_INF = np.float32(-np.inf)


def _prep_body(conf_ref, lx_ref, ly_ref, tb_ref,
               ms_ref, x1_ref, y1_ref, x2_ref, y2_ref, ar_ref, mk_ref):
  conf = conf_ref[...]
  lx = lx_ref[...]
  ly = ly_ref[...]
  t0 = tb_ref[0:1, 0:1]
  t1 = tb_ref[0:1, 1:2]
  t2 = tb_ref[0:1, 2:3]
  t3 = tb_ref[0:1, 3:4]
  cmax = jnp.max(conf, axis=0, keepdims=True)
  mask = cmax > np.float32(0.5)
  score = conf[0:1, :]
  ms_ref[...] = jnp.where(mask, score, NEG_INF)
  x1 = t0 * lx
  y1 = t1 * ly
  x2 = t2 * lx
  y2 = t3 * ly
  x1_ref[...] = x1
  y1_ref[...] = y1
  x2_ref[...] = x2
  y2_ref[...] = y2
  ar_ref[...] = (x2 - x1) * (y2 - y1)
  mk_ref[...] = mask.astype(jnp.float32)


_prep = pl.pallas_call(
    _prep_body,
    out_shape=[jax.ShapeDtypeStruct((1, NPAD), jnp.float32)] * 7,
)


@functools.lru_cache(maxsize=None)
def _make_sc_nms():
  mesh = plsc.VectorSubcoreMesh(
      core_axis_name="c", subcore_axis_name="s", num_cores=2,
      num_subcores=16)
  return functools.partial(
      pl.kernel,
      out_type=jax.ShapeDtypeStruct((NPAD,), jnp.float32),
      mesh=mesh,
      scratch_types=[pltpu.VMEM((NPAD,), jnp.float32) for _ in range(7)],
      compiler_params=pltpu.CompilerParams(needs_layout_passes=False),
  )(_sc_nms_body)


def _sc_nms_body(ms_hbm, x1_hbm, y1_hbm, x2_hbm, y2_hbm, ar_hbm, kept_hbm,
                 ms_v, x1_v, y1_v, x2_v, y2_v, ar_v, kept_v):
  wid = lax.axis_index("s") * 2 + lax.axis_index("c")

  @pl.when(wid == 0)
  def _():
    pltpu.sync_copy(ms_hbm, ms_v)
    pltpu.sync_copy(x1_hbm, x1_v)
    pltpu.sync_copy(y1_hbm, y1_v)
    pltpu.sync_copy(x2_hbm, x2_v)
    pltpu.sync_copy(y2_hbm, y2_v)
    pltpu.sync_copy(ar_hbm, ar_v)

    lanes = lax.iota(jnp.int32, 16)
    zero16 = jnp.zeros((16,), jnp.float32)
    neg16 = jnp.full((16,), NEG_INF)
    big = np.int32(1 << 30)

    def zero_body(j, carry):
      kept_v[pl.ds(j * 16, 16)] = zero16
      return carry

    lax.fori_loop(0, CHUNKS, zero_body, 0)

    def amax_body(j, carry):
      rm, ri, gi = carry
      c = ms_v[pl.ds(j * 16, 16)]
      upd = c > rm
      rm = jnp.where(upd, c, rm)
      ri = jnp.where(upd, gi, ri)
      return rm, ri, gi + 16

    rm, ri, _ = lax.fori_loop(0, CHUNKS, amax_body,
                              (neg16, jnp.zeros((16,), jnp.int32), lanes))
    m = jnp.max(rm)
    pidx = jnp.min(jnp.where(rm == m, ri, big))

    def cond(state):
      m, _ = state
      return m > NEG_INF

    def body(state):
      _, pidx = state
      pv = jnp.full((16,), pidx, jnp.int32)
      px1 = plsc.load_gather(x1_v, [pv])
      py1 = plsc.load_gather(y1_v, [pv])
      px2 = plsc.load_gather(x2_v, [pv])
      py2 = plsc.load_gather(y2_v, [pv])
      pa = plsc.load_gather(ar_v, [pv])
      lane0 = lanes == 0
      plsc.store_scatter(kept_v, [pv], jnp.ones((16,), jnp.float32),
                         mask=lane0)
      plsc.store_scatter(ms_v, [pv], neg16, mask=lane0)

      def sweep(j, carry):
        rm, ri, gi = carry
        sl = pl.ds(j * 16, 16)
        al = ms_v[sl]
        ax1 = x1_v[sl]
        ay1 = y1_v[sl]
        ax2 = x2_v[sl]
        ay2 = y2_v[sl]
        aa = ar_v[sl]
        xx1 = jnp.maximum(px1, ax1)
        yy1 = jnp.maximum(py1, ay1)
        xx2 = jnp.minimum(px2, ax2)
        yy2 = jnp.minimum(py2, ay2)
        w = jnp.maximum(xx2 - xx1, np.float32(0.0))
        h = jnp.maximum(yy2 - yy1, np.float32(0.0))
        inter = w * h
        iou = inter / (pa + aa - inter + np.float32(1e-12))
        al2 = jnp.where(iou > np.float32(0.5), neg16, al)
        ms_v[sl] = al2
        upd = al2 > rm
        rm = jnp.where(upd, al2, rm)
        ri = jnp.where(upd, gi, ri)
        return rm, ri, gi + 16

      rm, ri, _ = lax.fori_loop(0, CHUNKS, sweep,
                                (neg16, jnp.zeros((16,), jnp.int32), lanes))
      m2 = jnp.max(rm)
      pidx2 = jnp.min(jnp.where(rm == m2, ri, big))
      return m2, pidx2

    lax.while_loop(cond, body, (m, pidx))
    pltpu.sync_copy(kept_v, kept_hbm)


def _loss_body(maskr_ref, keptr_ref, kept_ref, conf_ref,
               x1_ref, y1_ref, x2_ref, y2_ref, tb_ref, out_ref):
  maskr = maskr_ref[...]
  keptr = keptr_ref[...]
  kept = kept_ref[...]
  conf = conf_ref[...]

  rows = maskr.shape[0]
  cols = maskr.shape[1]
  io_r = lax.broadcasted_iota(jnp.int32, (cols, cols), 0)
  io_c = lax.broadcasted_iota(jnp.int32, (cols, cols), 1)
  upper = (io_r <= io_c).astype(jnp.float32)
  within = lax.dot(maskr, upper,
                   preferred_element_type=jnp.float32)
  rowsum = within[:, cols - 1:cols]
  lo_r = lax.broadcasted_iota(jnp.int32, (rows, rows), 0)
  lo_c = lax.broadcasted_iota(jnp.int32, (rows, rows), 1)
  lower = (lo_c < lo_r).astype(jnp.float32)
  offs = lax.dot(lower, rowsum,
                 preferred_element_type=jnp.float32)
  ranks = within + offs - np.float32(1.0)
  num_positives = jnp.sum(keptr * ranks)

  keptb = kept > np.float32(0.5)
  mc = jnp.where(keptb, conf, NEG_INF)
  maxv = jnp.max(mc, axis=1, keepdims=True)
  colio = lax.broadcasted_iota(jnp.int32, (NCLS, NPAD), 1)
  idx = jnp.min(jnp.where(mc == maxv, colio, np.int32(1 << 30)),
                axis=1, keepdims=True)
  onehot = (colio == idx).astype(jnp.float32)

  mlx1 = jnp.sum(onehot * x1_ref[...], axis=1, keepdims=True)
  mly1 = jnp.sum(onehot * y1_ref[...], axis=1, keepdims=True)
  mlx2 = jnp.sum(onehot * x2_ref[...], axis=1, keepdims=True)
  mly2 = jnp.sum(onehot * y2_ref[...], axis=1, keepdims=True)

  def smooth_l1(d):
    ad = jnp.abs(d)
    return jnp.where(ad < np.float32(1.0),
                     np.float32(0.5) * d * d,
                     ad - np.float32(0.5))

  t0 = tb_ref[0:1, 0:1]
  t1 = tb_ref[0:1, 1:2]
  t2 = tb_ref[0:1, 2:3]
  t3 = tb_ref[0:1, 3:4]
  loc_loss = (jnp.sum(smooth_l1(mlx1 - t0)) +
              jnp.sum(smooth_l1(mly1 - t1)) +
              jnp.sum(smooth_l1(mlx2 - t2)) +
              jnp.sum(smooth_l1(mly2 - t3)))

  total = loc_loss / num_positives
  any_valid = jnp.max(maskr) > np.float32(0.0)
  has_keep = jnp.max(keptr) > np.float32(0.0)
  res = jnp.where(any_valid & has_keep, total, np.float32(0.001))
  out_ref[...] = jnp.full((1, 1), res, jnp.float32)


_loss = pl.pallas_call(
    _loss_body,
    out_shape=jax.ShapeDtypeStruct((1, 1), jnp.float32),
)


def kernel(loc, conf, target_boxes, target_labels):
  del target_labels
  confp = jnp.pad(conf.T, ((0, 0), (0, NPAD - N)))
  lxp = jnp.pad(loc[0, :, 0], (0, NPAD - N)).reshape(1, NPAD)
  lyp = jnp.pad(loc[0, :, 1], (0, NPAD - N)).reshape(1, NPAD)
  tb4 = target_boxes.reshape(1, 4)

  ms, x1, y1, x2, y2, ar, mk = _prep(confp, lxp, lyp, tb4)

  kept = _make_sc_nms()(ms.reshape(NPAD), x1.reshape(NPAD), y1.reshape(NPAD),
                        x2.reshape(NPAD), y2.reshape(NPAD), ar.reshape(NPAD))

  out = _loss(mk.reshape(40, 128), kept.reshape(40, 128),
              kept.reshape(1, NPAD), confp, x1, y1, x2, y2, tb4)
  return out[0, 0]


# speedup vs baseline: 767.5746x; 1.7088x over previous
import functools

import jax
import jax.numpy as jnp
import numpy as np
from jax import lax
from jax.experimental import pallas as pl
from jax.experimental.pallas import tpu as pltpu
from jax.experimental.pallas import tpu_sc as plsc

N = 5000
NPAD = 5120
NCLS = 21
CHUNKS = NPAD // 16
NEG_INF = np.float32(-np.inf)


def _prep_body(conf_ref, lx_ref, ly_ref, tb_ref,
               ms_ref, x1_ref, y1_ref, x2_ref, y2_ref, ar_ref, mk_ref):
  conf = conf_ref[...]
  lx = lx_ref[...]
  ly = ly_ref[...]
  t0 = tb_ref[0:1, 0:1]
  t1 = tb_ref[0:1, 1:2]
  t2 = tb_ref[0:1, 2:3]
  t3 = tb_ref[0:1, 3:4]
  cmax = jnp.max(conf, axis=0, keepdims=True)
  mask = cmax > np.float32(0.5)
  score = conf[0:1, :]
  ms_ref[...] = jnp.where(mask, score, NEG_INF)
  x1 = t0 * lx
  y1 = t1 * ly
  x2 = t2 * lx
  y2 = t3 * ly
  x1_ref[...] = x1
  y1_ref[...] = y1
  x2_ref[...] = x2
  y2_ref[...] = y2
  ar_ref[...] = (x2 - x1) * (y2 - y1)
  mk_ref[...] = mask.astype(jnp.float32)


_prep = pl.pallas_call(
    _prep_body,
    out_shape=[jax.ShapeDtypeStruct((1, NPAD), jnp.float32)] * 7,
)


P = NPAD // 16
CH_T = P // 16


@functools.lru_cache(maxsize=None)
def _make_sc_nms():
  mesh = plsc.VectorSubcoreMesh(
      core_axis_name="c", subcore_axis_name="s", num_cores=2,
      num_subcores=16)
  return functools.partial(
      pl.kernel,
      out_type=jax.ShapeDtypeStruct((NPAD,), jnp.float32),
      mesh=mesh,
      scratch_types=[pltpu.VMEM((NPAD,), jnp.float32) for _ in range(5)]
      + [pltpu.VMEM((P,), jnp.float32), pltpu.VMEM((P,), jnp.float32),
         pltpu.VMEM((16,), jnp.float32), pltpu.VMEM((128,), jnp.float32),
         pltpu.VMEM_SHARED((256,), jnp.float32)],
      compiler_params=pltpu.CompilerParams(needs_layout_passes=False),
  )(_sc_nms_body)


def _sc_nms_body(ms_hbm, x1_hbm, y1_hbm, x2_hbm, y2_hbm, ar_hbm, kept_hbm,
                 x1_v, y1_v, x2_v, y2_v, ar_v,
                 msl_v, keptl_v, pub_v, rd_v, shared_v):
  cid = lax.axis_index("c")
  sid = lax.axis_index("s")
  base = sid * P

  lanes = lax.iota(jnp.int32, 16)
  zero16 = jnp.zeros((16,), jnp.float32)
  neg16 = jnp.full((16,), NEG_INF)
  big = np.int32(1 << 30)
  lane0 = lanes == 0
  lane1 = lanes == 1

  pltpu.sync_copy(x1_hbm, x1_v)
  pltpu.sync_copy(y1_hbm, y1_v)
  pltpu.sync_copy(x2_hbm, x2_v)
  pltpu.sync_copy(y2_hbm, y2_v)
  pltpu.sync_copy(ar_hbm, ar_v)
  pltpu.sync_copy(ms_hbm.at[pl.ds(base, P)], msl_v)

  def zero_body(j, carry):
    keptl_v[pl.ds(j * 16, 16)] = zero16
    return carry

  lax.fori_loop(0, CH_T, zero_body, 0)

  def amax_body(j, carry):
    rm, ri = carry
    c = msl_v[pl.ds(j * 16, 16)]
    gi = base + j * 16 + lanes
    upd = c > rm
    rm = jnp.where(upd, c, rm)
    ri = jnp.where(upd, gi, ri)
    return rm, ri

  def local_reduce(rm, ri):
    m_w = jnp.max(rm)
    i_w = jnp.min(jnp.where(rm == m_w, ri, big))
    return m_w, i_w

  def publish(par, m_w, i_w):
    ivec = plsc.bitcast(jnp.full((16,), i_w, jnp.int32), jnp.float32)
    vec = jnp.where(lane0, jnp.full((16,), m_w, jnp.float32),
                    jnp.where(lane1, ivec, zero16))
    pub_v[...] = vec
    pltpu.sync_copy(pub_v.at[pl.ds(0, 8)],
                    shared_v.at[pl.ds(par * 128 + 8 * sid, 8)])

  def read_reduce(par):
    pltpu.sync_copy(shared_v.at[pl.ds(par * 128, 128)], rd_v)
    sc = plsc.load_gather(rd_v, [lanes * 8])
    ix = plsc.bitcast(plsc.load_gather(rd_v, [lanes * 8 + 1]), jnp.int32)
    m = jnp.max(sc)
    g = jnp.min(jnp.where(sc == m, ix, big))
    return m, g

  rm, ri = lax.fori_loop(0, CH_T, amax_body,
                         (neg16, jnp.zeros((16,), jnp.int32)))
  m_w, i_w = local_reduce(rm, ri)
  publish(0, m_w, i_w)
  plsc.subcore_barrier()
  m0, g0 = read_reduce(0)

  def cond(state):
    _, m, _ = state
    return m > NEG_INF

  def body(state):
    par, _, g = state
    pv = jnp.full((16,), g, jnp.int32)
    px1 = plsc.load_gather(x1_v, [pv])
    py1 = plsc.load_gather(y1_v, [pv])
    px2 = plsc.load_gather(x2_v, [pv])
    py2 = plsc.load_gather(y2_v, [pv])
    pa = plsc.load_gather(ar_v, [pv])
    off = g - base
    inrange = (off >= 0) & (off < P)
    offc = jnp.clip(off, 0, P - 1)
    ov = jnp.full((16,), offc, jnp.int32)
    wmask = jnp.logical_and(lane0, inrange)
    plsc.store_scatter(keptl_v, [ov], jnp.ones((16,), jnp.float32),
                       mask=wmask)
    plsc.store_scatter(msl_v, [ov], neg16, mask=wmask)

    def sweep(j, carry):
      rm, ri = carry
      gb = base + j * 16
      sll = pl.ds(j * 16, 16)
      al = msl_v[sll]
      ax1 = x1_v[pl.ds(gb, 16)]
      ay1 = y1_v[pl.ds(gb, 16)]
      ax2 = x2_v[pl.ds(gb, 16)]
      ay2 = y2_v[pl.ds(gb, 16)]
      aa = ar_v[pl.ds(gb, 16)]
      xx1 = jnp.maximum(px1, ax1)
      yy1 = jnp.maximum(py1, ay1)
      xx2 = jnp.minimum(px2, ax2)
      yy2 = jnp.minimum(py2, ay2)
      w = jnp.maximum(xx2 - xx1, np.float32(0.0))
      h = jnp.maximum(yy2 - yy1, np.float32(0.0))
      inter = w * h
      iou = inter / (pa + aa - inter + np.float32(1e-12))
      al2 = jnp.where(iou > np.float32(0.5), neg16, al)
      msl_v[sll] = al2
      gi = gb + lanes
      upd = al2 > rm
      rm = jnp.where(upd, al2, rm)
      ri = jnp.where(upd, gi, ri)
      return rm, ri

    rm, ri = lax.fori_loop(0, CH_T, sweep,
                           (neg16, jnp.zeros((16,), jnp.int32)))
    m_w, i_w = local_reduce(rm, ri)
    par2 = 1 - par
    publish(par2, m_w, i_w)
    plsc.subcore_barrier()
    m2, g2 = read_reduce(par2)
    return par2, m2, g2

  lax.while_loop(cond, body, (np.int32(0), m0, g0))

  @pl.when(cid == 0)
  def _():
    pltpu.sync_copy(keptl_v, kept_hbm.at[pl.ds(base, P)])


def _loss_body(maskr_ref, keptr_ref, kept_ref, conf_ref,
               x1_ref, y1_ref, x2_ref, y2_ref, tb_ref, out_ref):
  maskr = maskr_ref[...]
  keptr = keptr_ref[...]
  kept = kept_ref[...]
  conf = conf_ref[...]

  rows = maskr.shape[0]
  cols = maskr.shape[1]
  io_r = lax.broadcasted_iota(jnp.int32, (cols, cols), 0)
  io_c = lax.broadcasted_iota(jnp.int32, (cols, cols), 1)
  upper = (io_r <= io_c).astype(jnp.float32)
  within = lax.dot(maskr, upper,
                   preferred_element_type=jnp.float32)
  rowsum = within[:, cols - 1:cols]
  lo_r = lax.broadcasted_iota(jnp.int32, (rows, rows), 0)
  lo_c = lax.broadcasted_iota(jnp.int32, (rows, rows), 1)
  lower = (lo_c < lo_r).astype(jnp.float32)
  offs = lax.dot(lower, rowsum,
                 preferred_element_type=jnp.float32)
  ranks = within + offs - np.float32(1.0)
  num_positives = jnp.sum(keptr * ranks)

  keptb = kept > np.float32(0.5)
  mc = jnp.where(keptb, conf, NEG_INF)
  maxv = jnp.max(mc, axis=1, keepdims=True)
  colio = lax.broadcasted_iota(jnp.int32, (NCLS, NPAD), 1)
  idx = jnp.min(jnp.where(mc == maxv, colio, np.int32(1 << 30)),
                axis=1, keepdims=True)
  onehot = (colio == idx).astype(jnp.float32)

  mlx1 = jnp.sum(onehot * x1_ref[...], axis=1, keepdims=True)
  mly1 = jnp.sum(onehot * y1_ref[...], axis=1, keepdims=True)
  mlx2 = jnp.sum(onehot * x2_ref[...], axis=1, keepdims=True)
  mly2 = jnp.sum(onehot * y2_ref[...], axis=1, keepdims=True)

  def smooth_l1(d):
    ad = jnp.abs(d)
    return jnp.where(ad < np.float32(1.0),
                     np.float32(0.5) * d * d,
                     ad - np.float32(0.5))

  t0 = tb_ref[0:1, 0:1]
  t1 = tb_ref[0:1, 1:2]
  t2 = tb_ref[0:1, 2:3]
  t3 = tb_ref[0:1, 3:4]
  loc_loss = (jnp.sum(smooth_l1(mlx1 - t0)) +
              jnp.sum(smooth_l1(mly1 - t1)) +
              jnp.sum(smooth_l1(mlx2 - t2)) +
              jnp.sum(smooth_l1(mly2 - t3)))

  total = loc_loss / num_positives
  any_valid = jnp.max(maskr) > np.float32(0.0)
  has_keep = jnp.max(keptr) > np.float32(0.0)
  res = jnp.where(any_valid & has_keep, total, np.float32(0.001))
  out_ref[...] = jnp.full((1, 1), res, jnp.float32)


_loss = pl.pallas_call(
    _loss_body,
    out_shape=jax.ShapeDtypeStruct((1, 1), jnp.float32),
)


def kernel(loc, conf, target_boxes, target_labels):
  del target_labels
  confp = jnp.pad(conf.T, ((0, 0), (0, NPAD - N)))
  lxp = jnp.pad(loc[0, :, 0], (0, NPAD - N)).reshape(1, NPAD)
  lyp = jnp.pad(loc[0, :, 1], (0, NPAD - N)).reshape(1, NPAD)
  tb4 = target_boxes.reshape(1, 4)

  ms, x1, y1, x2, y2, ar, mk = _prep(confp, lxp, lyp, tb4)

  kept = _make_sc_nms()(ms.reshape(NPAD), x1.reshape(NPAD), y1.reshape(NPAD),
                        x2.reshape(NPAD), y2.reshape(NPAD), ar.reshape(NPAD))

  out = _loss(mk.reshape(40, 128), kept.reshape(40, 128),
              kept.reshape(1, NPAD), confp, x1, y1, x2, y2, tb4)
  return out[0, 0]


# speedup vs baseline: 864.6692x; 1.1265x over previous
import functools

import jax
import jax.numpy as jnp
import numpy as np
from jax import lax
from jax.experimental import pallas as pl
from jax.experimental.pallas import tpu as pltpu
from jax.experimental.pallas import tpu_sc as plsc

N = 5000
NPAD = 5120
NCLS = 21
CHUNKS = NPAD // 16
NEG_INF = np.float32(-np.inf)


def _prep_body(conf_ref, lx_ref, ly_ref, tb_ref,
               ms_ref, x1_ref, y1_ref, x2_ref, y2_ref, ar_ref, mk_ref):
  conf = conf_ref[...]
  lx = lx_ref[...]
  ly = ly_ref[...]
  t0 = tb_ref[0:1, 0:1]
  t1 = tb_ref[0:1, 1:2]
  t2 = tb_ref[0:1, 2:3]
  t3 = tb_ref[0:1, 3:4]
  cmax = jnp.max(conf, axis=0, keepdims=True)
  mask = cmax > np.float32(0.5)
  score = conf[0:1, :]
  ms_ref[...] = jnp.where(mask, score, NEG_INF)
  x1 = t0 * lx
  y1 = t1 * ly
  x2 = t2 * lx
  y2 = t3 * ly
  x1_ref[...] = x1
  y1_ref[...] = y1
  x2_ref[...] = x2
  y2_ref[...] = y2
  ar_ref[...] = (x2 - x1) * (y2 - y1)
  mk_ref[...] = mask.astype(jnp.float32)


_prep = pl.pallas_call(
    _prep_body,
    out_shape=[jax.ShapeDtypeStruct((1, NPAD), jnp.float32)] * 7,
)


P = NPAD // 16
CH_T = P // 16


@functools.lru_cache(maxsize=None)
def _make_sc_nms():
  mesh = plsc.VectorSubcoreMesh(
      core_axis_name="c", subcore_axis_name="s", num_cores=2,
      num_subcores=16)
  return functools.partial(
      pl.kernel,
      out_type=jax.ShapeDtypeStruct((NPAD,), jnp.float32),
      mesh=mesh,
      scratch_types=[pltpu.VMEM((NPAD,), jnp.float32) for _ in range(5)]
      + [pltpu.VMEM((P,), jnp.float32), pltpu.VMEM((P,), jnp.float32),
         pltpu.VMEM((16,), jnp.float32), pltpu.VMEM((128,), jnp.float32),
         pltpu.VMEM((128,), jnp.float32),
         pltpu.VMEM_SHARED((256,), jnp.float32)],
      compiler_params=pltpu.CompilerParams(needs_layout_passes=False),
  )(_sc_nms_body)


def _sc_nms_body(ms_hbm, x1_hbm, y1_hbm, x2_hbm, y2_hbm, ar_hbm, kept_hbm,
                 x1_v, y1_v, x2_v, y2_v, ar_v,
                 msl_v, keptl_v, pub_v, rd_v, acc_v, shared_v):
  cid = lax.axis_index("c")
  sid = lax.axis_index("s")
  base = sid * P

  lanes = lax.iota(jnp.int32, 16)
  zero16 = jnp.zeros((16,), jnp.float32)
  izero16 = jnp.zeros((16,), jnp.int32)
  ones16 = jnp.ones((16,), jnp.float32)
  false16 = jnp.zeros((16,), jnp.bool_)
  neg16 = jnp.full((16,), NEG_INF)
  big = np.int32(1 << 30)
  bigv = jnp.full((16,), big, jnp.int32)
  lane0 = lanes == 0
  lane1 = lanes == 1
  lane2 = lanes == 2
  lane3 = lanes == 3

  pltpu.sync_copy(x1_hbm, x1_v)
  pltpu.sync_copy(y1_hbm, y1_v)
  pltpu.sync_copy(x2_hbm, x2_v)
  pltpu.sync_copy(y2_hbm, y2_v)
  pltpu.sync_copy(ar_hbm, ar_v)
  pltpu.sync_copy(ms_hbm.at[pl.ds(base, P)], msl_v)

  def zero_body(j, carry):
    keptl_v[pl.ds(j * 16, 16)] = zero16
    return carry

  lax.fori_loop(0, CH_T, zero_body, 0)

  def top2_publish_read(par):
    def am(j, carry):
      rm1, ri1, rm2, ri2 = carry
      c = msl_v[pl.ds(j * 16, 16)]
      gi = base + j * 16 + lanes
      gt1 = c > rm1
      gt2 = c > rm2
      rm2n = jnp.where(gt1, rm1, jnp.where(gt2, c, rm2))
      ri2n = jnp.where(gt1, ri1, jnp.where(gt2, gi, ri2))
      rm1n = jnp.where(gt1, c, rm1)
      ri1n = jnp.where(gt1, gi, ri1)
      return rm1n, ri1n, rm2n, ri2n

    rm1, ri1, rm2, ri2 = lax.fori_loop(
        0, CH_T, am, (neg16, izero16, neg16, izero16))
    s1 = jnp.max(rm1)
    i1 = jnp.min(jnp.where(rm1 == s1, ri1, bigv))
    wl = (rm1 == s1) & (ri1 == i1)
    rm1b = jnp.where(wl, rm2, rm1)
    ri1b = jnp.where(wl, ri2, ri1)
    s2 = jnp.max(rm1b)
    i2 = jnp.min(jnp.where(rm1b == s2, ri1b, bigv))

    i1f = plsc.bitcast(jnp.full((16,), i1, jnp.int32), jnp.float32)
    i2f = plsc.bitcast(jnp.full((16,), i2, jnp.int32), jnp.float32)
    vec = jnp.where(lane0, jnp.full((16,), s1, jnp.float32),
                    jnp.where(lane1, i1f,
                              jnp.where(lane2,
                                        jnp.full((16,), s2, jnp.float32),
                                        jnp.where(lane3, i2f, zero16))))
    pub_v[...] = vec
    pltpu.sync_copy(pub_v.at[pl.ds(0, 8)],
                    shared_v.at[pl.ds(par * 128 + 8 * sid, 8)])
    plsc.subcore_barrier()
    pltpu.sync_copy(shared_v.at[pl.ds(par * 128, 128)], rd_v)
    s1s = plsc.load_gather(rd_v, [lanes * 8])
    i1s = plsc.bitcast(plsc.load_gather(rd_v, [lanes * 8 + 1]), jnp.int32)
    s2s = plsc.load_gather(rd_v, [lanes * 8 + 2])
    i2s = plsc.bitcast(plsc.load_gather(rd_v, [lanes * 8 + 3]), jnp.int32)
    return s1s, i1s, s2s, i2s

  pool0 = top2_publish_read(0)
  mg0 = jnp.max(pool0[0])

  def cond(state):
    return state[1] > NEG_INF

  def body(state):
    par, _, s1s, i1s, s2s, i2s = state

    def bcond(st):
      return jnp.logical_not(st[0])

    def bbody(st):
      (stop, steps, u1, u2, am, aI,
       ax1, ay1, ax2, ay2, aa) = st
      c1v = jnp.where(u1, s1s, neg16)
      c2v = jnp.where(u2, s2s, neg16)
      m = jnp.max(jnp.maximum(c1v, c2v))
      candv = jnp.minimum(jnp.where(u1 & (s1s == m), i1s, bigv),
                          jnp.where(u2 & (s2s == m), i2s, bigv))
      ci = jnp.min(candv)
      xb = jnp.max(jnp.where(u2, neg16, s2s))
      go = (m > NEG_INF) & (m > xb)

      pv = jnp.full((16,), ci, jnp.int32)
      cx1 = plsc.load_gather(x1_v, [pv])
      cy1 = plsc.load_gather(y1_v, [pv])
      cx2 = plsc.load_gather(x2_v, [pv])
      cy2 = plsc.load_gather(y2_v, [pv])
      ca = plsc.load_gather(ar_v, [pv])
      xx1 = jnp.maximum(ax1, cx1)
      yy1 = jnp.maximum(ay1, cy1)
      xx2 = jnp.minimum(ax2, cx2)
      yy2 = jnp.minimum(ay2, cy2)
      w = jnp.maximum(xx2 - xx1, np.float32(0.0))
      h = jnp.maximum(yy2 - yy1, np.float32(0.0))
      inter = w * h
      iou = inter / (aa + ca - inter + np.float32(1e-12))
      suppv = am & (iou > np.float32(0.5))
      suppb = plsc.all_reduce_population_count(suppv) > 0

      u1n = jnp.where(go, u1 & (i1s != ci), u1)
      u2n = jnp.where(go, u2 & (i2s != ci), u2)
      slot = plsc.all_reduce_ffs(jnp.logical_not(am))
      sel = (lanes == slot) & jnp.logical_not(suppb) & go
      amn = am | sel
      aIn = jnp.where(sel, pv, aI)
      ax1n = jnp.where(sel, cx1, ax1)
      ay1n = jnp.where(sel, cy1, ay1)
      ax2n = jnp.where(sel, cx2, ax2)
      ay2n = jnp.where(sel, cy2, ay2)
      aan = jnp.where(sel, ca, aa)
      steps2 = steps + 1
      stop2 = jnp.logical_not(go) | (steps2 >= 16)
      return (stop2, steps2, u1n, u2n, amn, aIn,
              ax1n, ay1n, ax2n, ay2n, aan)

    init = (False, np.int32(0),
            jnp.ones((16,), jnp.bool_), jnp.ones((16,), jnp.bool_),
            false16, izero16, zero16, zero16, zero16, zero16, zero16)
    (_, _, _, _, am, aI, ax1, ay1, ax2, ay2, aa) = lax.while_loop(
        bcond, bbody, init)

    offv = aI - base
    wm = am & (offv >= 0) & (offv < P)
    offc = jnp.clip(offv, 0, P - 1)
    plsc.store_scatter(keptl_v, [offc], ones16, mask=wm)
    plsc.store_scatter(msl_v, [offc], neg16, mask=wm)

    acc_v[pl.ds(0, 16)] = ax1
    acc_v[pl.ds(16, 16)] = ay1
    acc_v[pl.ds(32, 16)] = ax2
    acc_v[pl.ds(48, 16)] = ay2
    acc_v[pl.ds(64, 16)] = aa
    cnt = jnp.max(plsc.all_reduce_population_count(am))

    def per_pivot(k, carry):
      kv = jnp.full((16,), k, jnp.int32)
      px1 = plsc.load_gather(acc_v, [kv])
      py1 = plsc.load_gather(acc_v, [kv + 16])
      px2 = plsc.load_gather(acc_v, [kv + 32])
      py2 = plsc.load_gather(acc_v, [kv + 48])
      pa = plsc.load_gather(acc_v, [kv + 64])

      def sweep(j, c2):
        sll = pl.ds(j * 16, 16)
        gb = base + j * 16
        al = msl_v[sll]
        bx1 = x1_v[pl.ds(gb, 16)]
        by1 = y1_v[pl.ds(gb, 16)]
        bx2 = x2_v[pl.ds(gb, 16)]
        by2 = y2_v[pl.ds(gb, 16)]
        ba = ar_v[pl.ds(gb, 16)]
        xx1 = jnp.maximum(px1, bx1)
        yy1 = jnp.maximum(py1, by1)
        xx2 = jnp.minimum(px2, bx2)
        yy2 = jnp.minimum(py2, by2)
        w = jnp.maximum(xx2 - xx1, np.float32(0.0))
        h = jnp.maximum(yy2 - yy1, np.float32(0.0))
        inter = w * h
        iou = inter / (pa + ba - inter + np.float32(1e-12))
        msl_v[sll] = jnp.where(iou > np.float32(0.5), neg16, al)
        return c2

      lax.fori_loop(0, CH_T, sweep, 0)
      return carry

    lax.fori_loop(0, cnt, per_pivot, 0)

    par2 = 1 - par
    s1s2, i1s2, s2s2, i2s2 = top2_publish_read(par2)
    mg = jnp.max(s1s2)
    return par2, mg, s1s2, i1s2, s2s2, i2s2

  lax.while_loop(cond, body, (np.int32(0), mg0) + pool0)

  @pl.when(cid == 0)
  def _():
    pltpu.sync_copy(keptl_v, kept_hbm.at[pl.ds(base, P)])


def _loss_body(maskr_ref, keptr_ref, kept_ref, conf_ref,
               x1_ref, y1_ref, x2_ref, y2_ref, tb_ref, out_ref):
  maskr = maskr_ref[...]
  keptr = keptr_ref[...]
  kept = kept_ref[...]
  conf = conf_ref[...]

  rows = maskr.shape[0]
  cols = maskr.shape[1]
  io_r = lax.broadcasted_iota(jnp.int32, (cols, cols), 0)
  io_c = lax.broadcasted_iota(jnp.int32, (cols, cols), 1)
  upper = (io_r <= io_c).astype(jnp.float32)
  within = lax.dot(maskr, upper,
                   preferred_element_type=jnp.float32)
  rowsum = within[:, cols - 1:cols]
  lo_r = lax.broadcasted_iota(jnp.int32, (rows, rows), 0)
  lo_c = lax.broadcasted_iota(jnp.int32, (rows, rows), 1)
  lower = (lo_c < lo_r).astype(jnp.float32)
  offs = lax.dot(lower, rowsum,
                 preferred_element_type=jnp.float32)
  ranks = within + offs - np.float32(1.0)
  num_positives = jnp.sum(keptr * ranks)

  keptb = kept > np.float32(0.5)
  mc = jnp.where(keptb, conf, NEG_INF)
  maxv = jnp.max(mc, axis=1, keepdims=True)
  colio = lax.broadcasted_iota(jnp.int32, (NCLS, NPAD), 1)
  idx = jnp.min(jnp.where(mc == maxv, colio, np.int32(1 << 30)),
                axis=1, keepdims=True)
  onehot = (colio == idx).astype(jnp.float32)

  mlx1 = jnp.sum(onehot * x1_ref[...], axis=1, keepdims=True)
  mly1 = jnp.sum(onehot * y1_ref[...], axis=1, keepdims=True)
  mlx2 = jnp.sum(onehot * x2_ref[...], axis=1, keepdims=True)
  mly2 = jnp.sum(onehot * y2_ref[...], axis=1, keepdims=True)

  def smooth_l1(d):
    ad = jnp.abs(d)
    return jnp.where(ad < np.float32(1.0),
                     np.float32(0.5) * d * d,
                     ad - np.float32(0.5))

  t0 = tb_ref[0:1, 0:1]
  t1 = tb_ref[0:1, 1:2]
  t2 = tb_ref[0:1, 2:3]
  t3 = tb_ref[0:1, 3:4]
  loc_loss = (jnp.sum(smooth_l1(mlx1 - t0)) +
              jnp.sum(smooth_l1(mly1 - t1)) +
              jnp.sum(smooth_l1(mlx2 - t2)) +
              jnp.sum(smooth_l1(mly2 - t3)))

  total = loc_loss / num_positives
  any_valid = jnp.max(maskr) > np.float32(0.0)
  has_keep = jnp.max(keptr) > np.float32(0.0)
  res = jnp.where(any_valid & has_keep, total, np.float32(0.001))
  out_ref[...] = jnp.full((1, 1), res, jnp.float32)


_loss = pl.pallas_call(
    _loss_body,
    out_shape=jax.ShapeDtypeStruct((1, 1), jnp.float32),
)


def kernel(loc, conf, target_boxes, target_labels):
  del target_labels
  confp = jnp.pad(conf.T, ((0, 0), (0, NPAD - N)))
  lxp = jnp.pad(loc[0, :, 0], (0, NPAD - N)).reshape(1, NPAD)
  lyp = jnp.pad(loc[0, :, 1], (0, NPAD - N)).reshape(1, NPAD)
  tb4 = target_boxes.reshape(1, 4)

  ms, x1, y1, x2, y2, ar, mk = _prep(confp, lxp, lyp, tb4)

  kept = _make_sc_nms()(ms.reshape(NPAD), x1.reshape(NPAD), y1.reshape(NPAD),
                        x2.reshape(NPAD), y2.reshape(NPAD), ar.reshape(NPAD))

  out = _loss(mk.reshape(40, 128), kept.reshape(40, 128),
              kept.reshape(1, NPAD), confp, x1, y1, x2, y2, tb4)
  return out[0, 0]
